# Initial kernel scaffold; baseline (speedup 1.0000x reference)
import jax
import jax.numpy as jnp
from jax.experimental import pallas as pl


def kernel(net_out, target, max_positiones):
    raise NotImplementedError("write your pallas kernel here")


# speedup vs baseline: 24.5028x; 24.5028x over previous
import functools

import jax
import jax.numpy as jnp
from jax.experimental import pallas as pl
from jax.experimental.pallas import tpu as pltpu

_B, _C, _H, _W = 4, 4, 384, 384
_N = _H * _W
_K = _N * 10 // 100
_ROWS = _N // 128
_NSLICES = _B * _C
_HI_BITS = 0x41600000


def _slice_body(no_ref, t_ref, mp_ref, out_ref, res_ref, bits_ref):
    no = no_ref[0]
    t = t_ref[0]
    res = -(t * jnp.log(no))
    res_ref[...] = res
    bits_ref[...] = jax.lax.bitcast_convert_type(res, jnp.int32)
    max_t = jnp.max(t)
    max_p = jnp.max(mp_ref[0])

    def step(_, carry):
        lo, hi = carry
        mid = lo + (hi - lo) // 2
        cnt = jnp.sum((bits_ref[...] >= mid).astype(jnp.int32))
        ge = cnt >= _K
        return (jnp.where(ge, mid, lo), jnp.where(ge, hi, mid))

    lo, hi = jax.lax.fori_loop(0, 31, step, (jnp.int32(0), jnp.int32(_HI_BITS)))
    bits = bits_ref[...]
    gt = bits > lo
    cnt_gt = jnp.sum(gt.astype(jnp.int32))
    sum_gt = jnp.sum(jnp.where(gt, res_ref[...], 0.0))
    pivot_val = jax.lax.bitcast_convert_type(lo, jnp.float32)
    loss = (sum_gt + (_K - cnt_gt).astype(jnp.float32) * pivot_val) / _K
    skip = (max_t == 0.0) & (max_p == 0.0)
    out_ref[...] = jnp.full((1, 1, 128), jnp.where(skip, 0.0, loss), jnp.float32)


@jax.jit
def kernel(net_out, target, max_positiones):
    shape3 = (_NSLICES, _ROWS, 128)
    no = net_out.reshape(shape3)
    t = target.reshape(shape3)
    mp = max_positiones.reshape(shape3)
    in_spec = pl.BlockSpec((1, _ROWS, 128), lambda i: (i, 0, 0))
    per = pl.pallas_call(
        _slice_body,
        grid=(_NSLICES,),
        in_specs=[in_spec, in_spec, in_spec],
        out_specs=pl.BlockSpec((1, 1, 128), lambda i: (i, 0, 0)),
        out_shape=jax.ShapeDtypeStruct((_NSLICES, 1, 128), jnp.float32),
        scratch_shapes=[
            pltpu.VMEM((_ROWS, 128), jnp.float32),
            pltpu.VMEM((_ROWS, 128), jnp.int32),
        ],
    )(no, t, mp)
    per = per[:, 0, 0].reshape(_B, _C)
    counts = jnp.count_nonzero(per, axis=1)
    img_losses = per.sum(axis=1) / counts
    return img_losses.sum() / _B
